# baseline (speedup 1.0000x reference)
import functools

import jax
import jax.numpy as jnp
from jax import lax
from jax.experimental import pallas as pl
from jax.experimental.pallas import tpu as pltpu


def _pooler_body(x_hbm, w_hbm, b_ref, o_hbm, x_vmem, o_vmem, w_vmem, xsems, osems, wsem,
                 *, sizes):
    offs = [0]
    for s in sizes:
        offs.append(offs[-1] + s)

    w_cp = pltpu.make_async_copy(w_hbm, w_vmem, wsem)
    w_cp.start()
    x_cps = []
    for c, s in enumerate(sizes):
        cp = pltpu.make_async_copy(
            x_hbm.at[pl.ds(offs[c], s), 0, :],
            x_vmem.at[pl.ds(offs[c], s), :],
            xsems.at[c])
        cp.start()
        x_cps.append(cp)

    w_cp.wait()
    o_cps = []
    for c, s in enumerate(sizes):
        x_cps[c].wait()
        y = lax.dot_general(
            x_vmem[pl.ds(offs[c], s), :],
            w_vmem[...],
            dimension_numbers=(((1,), (1,)), ((), ())),
            preferred_element_type=jnp.float32,
        )
        o_vmem[pl.ds(offs[c], s), :] = jnp.tanh(y + b_ref[...])
        cp = pltpu.make_async_copy(
            o_vmem.at[pl.ds(offs[c], s), :],
            o_hbm.at[pl.ds(offs[c], s), :],
            osems.at[c])
        cp.start()
        o_cps.append(cp)
    for cp in o_cps:
        cp.wait()


def kernel(x, weight, bias, *, sizes=None):
    B, S, D = x.shape
    assert weight.shape == (D, D) and bias.shape == (D,)
    if sizes is None:
        sizes = (B // 2, B - B // 2)
    assert sum(sizes) == B

    b2d = bias.reshape(1, D).astype(jnp.float32)

    cost = pl.CostEstimate(
        flops=2 * B * D * D,
        transcendentals=B * D,
        bytes_accessed=(D * D * jnp.dtype(weight.dtype).itemsize
                        + B * D * jnp.dtype(x.dtype).itemsize
                        + D * 4
                        + B * D * jnp.dtype(x.dtype).itemsize),
    )

    return pl.pallas_call(
        functools.partial(_pooler_body, sizes=tuple(sizes)),
        out_shape=jax.ShapeDtypeStruct((B, D), x.dtype),
        grid=(1,),
        in_specs=[
            pl.BlockSpec(memory_space=pl.ANY),
            pl.BlockSpec(memory_space=pl.ANY),
            pl.BlockSpec((1, D), lambda b: (0, 0)),
        ],
        out_specs=pl.BlockSpec(memory_space=pl.ANY),
        scratch_shapes=[
            pltpu.VMEM((B, D), jnp.float32),
            pltpu.VMEM((B, D), jnp.float32),
            pltpu.VMEM((D, D), jnp.bfloat16),
            pltpu.SemaphoreType.DMA((len(sizes),)),
            pltpu.SemaphoreType.DMA((len(sizes),)),
            pltpu.SemaphoreType.DMA,
        ],
        compiler_params=pltpu.CompilerParams(
            dimension_semantics=("arbitrary",),
            vmem_limit_bytes=48 * 1024 * 1024,
        ),
        cost_estimate=cost,
    )(x, weight, b2d)


# speedup vs baseline: 1.0499x; 1.0499x over previous
import functools

import jax
import jax.numpy as jnp
from jax import lax
from jax.experimental import pallas as pl
from jax.experimental.pallas import tpu as pltpu


def _pooler_body(x_hbm, w_ref, b_ref, o_hbm, x_vmem, o_vmem, xsems, osems,
                 *, sizes):
    offs = [0]
    for s in sizes:
        offs.append(offs[-1] + s)

    x_cps = []
    for c, s in enumerate(sizes):
        cp = pltpu.make_async_copy(
            x_hbm.at[pl.ds(offs[c], s), 0, :],
            x_vmem.at[pl.ds(offs[c], s), :],
            xsems.at[c])
        cp.start()
        x_cps.append(cp)

    o_cps = []
    for c, s in enumerate(sizes):
        x_cps[c].wait()
        y = lax.dot_general(
            x_vmem[pl.ds(offs[c], s), :],
            w_ref[...],
            dimension_numbers=(((1,), (1,)), ((), ())),
            preferred_element_type=jnp.float32,
        )
        o_vmem[pl.ds(offs[c], s), :] = jnp.tanh(y + b_ref[...])
        cp = pltpu.make_async_copy(
            o_vmem.at[pl.ds(offs[c], s), :],
            o_hbm.at[pl.ds(offs[c], s), :],
            osems.at[c])
        cp.start()
        o_cps.append(cp)
    for cp in o_cps:
        cp.wait()


def kernel(x, weight, bias, *, sizes=None):
    B, S, D = x.shape
    assert weight.shape == (D, D) and bias.shape == (D,)
    if sizes is None:
        sizes = (B // 2, B - B // 2)
    assert sum(sizes) == B

    b2d = bias.reshape(1, D).astype(jnp.float32)

    cost = pl.CostEstimate(
        flops=2 * B * D * D,
        transcendentals=B * D,
        bytes_accessed=(D * D * jnp.dtype(weight.dtype).itemsize
                        + B * D * jnp.dtype(x.dtype).itemsize
                        + D * 4
                        + B * D * jnp.dtype(x.dtype).itemsize),
    )

    return pl.pallas_call(
        functools.partial(_pooler_body, sizes=tuple(sizes)),
        out_shape=jax.ShapeDtypeStruct((B, D), x.dtype),
        in_specs=[
            pl.BlockSpec(memory_space=pl.ANY),
            pl.BlockSpec(memory_space=pltpu.MemorySpace.VMEM),
            pl.BlockSpec(memory_space=pltpu.MemorySpace.VMEM),
        ],
        out_specs=pl.BlockSpec(memory_space=pl.ANY),
        scratch_shapes=[
            pltpu.VMEM((B, D), jnp.float32),
            pltpu.VMEM((B, D), jnp.float32),
            pltpu.SemaphoreType.DMA((len(sizes),)),
            pltpu.SemaphoreType.DMA((len(sizes),)),
        ],
        compiler_params=pltpu.CompilerParams(
            vmem_limit_bytes=48 * 1024 * 1024,
        ),
        cost_estimate=cost,
    )(x, weight, b2d)
